# baseline (speedup 1.0000x reference)
import functools

import jax
import jax.numpy as jnp
import numpy as np
from jax import lax
from jax.experimental import pallas as pl
from jax.experimental.pallas import tpu as pltpu
from jax.experimental.pallas import tpu_sc as plsc

_B, _N = 256, 64
_D = 256
_EF = 64
_RUP = 5.0

_MEANS = np.linspace(0.0, _RUP, _EF, dtype=np.float32)
_DELTA = np.float32(_MEANS[1] - _MEANS[0])
_COEFF = float(np.float32(0.5) / (_DELTA * _DELTA))


_NW = 32
_TOTAL = _B * _N
_ROWS_W = _TOTAL // _NW
_CH = 128
_NCH = _ROWS_W // _CH


@functools.lru_cache(maxsize=1)
def _make_sc_gather():
    info = plsc.get_sparse_core_info()
    nc, ns = info.num_cores, info.num_subcores
    nw = nc * ns
    assert nw == _NW
    rows_w, ch, nch = _ROWS_W, _CH, _NCH

    mesh = plsc.VectorSubcoreMesh(core_axis_name="c", subcore_axis_name="s")

    @functools.partial(
        pl.kernel,
        mesh=mesh,
        out_type=jax.ShapeDtypeStruct((_TOTAL, _D), jnp.float32),
        scratch_types=[
            pltpu.VMEM((nch, ch), jnp.int32),
            pltpu.VMEM((ch, _D), jnp.float32),
            pltpu.VMEM((ch, _D), jnp.float32),
            pltpu.VMEM((ch, _D), jnp.float32),
            pltpu.SemaphoreType.DMA,
            pltpu.SemaphoreType.DMA,
            pltpu.SemaphoreType.DMA,
            pltpu.SemaphoreType.DMA,
            pltpu.SemaphoreType.DMA,
            pltpu.SemaphoreType.DMA,
        ],
    )
    def sc_gather(table_hbm, idx_hbm, out_hbm, idx_v, buf0, buf1, buf2,
                  gs0, gs1, gs2, os0, os1, os2):
        wid = lax.axis_index("s") * nc + lax.axis_index("c")
        base = wid * rows_w
        pltpu.sync_copy(idx_hbm.at[wid], idx_v)
        bufs = (buf0, buf1, buf2)
        gsems = (gs0, gs1, gs2)
        osems = (os0, os1, os2)
        nbuf = 3
        gathers = [None] * nch
        outs = [None] * nch
        for c in range(min(nbuf, nch)):
            gathers[c] = pltpu.async_copy(
                table_hbm.at[idx_v.at[c]], bufs[c % nbuf], gsems[c % nbuf]
            )
        for c in range(nch):
            gathers[c].wait()
            outs[c] = pltpu.async_copy(
                bufs[c % nbuf], out_hbm.at[pl.ds(base + c * ch, ch)],
                osems[c % nbuf],
            )
            nxt = c + nbuf
            if nxt < nch:
                outs[nxt - nbuf].wait()
                gathers[nxt] = pltpu.async_copy(
                    table_hbm.at[idx_v.at[nxt]], bufs[nxt % nbuf], gsems[nxt % nbuf]
                )
        for c in range(max(0, nch - nbuf), nch):
            outs[c].wait()

    return sc_gather


_C2 = float(-np.float64(_COEFF) * np.log2(np.e))


_JS = 2
_NJ = _N // _JS


def _ef_body(prow_ref, pall_ref, means_ref, out_ref):
    row = prow_ref[0]
    pall = pall_ref[0]

    dx = pall - row[:, None, :]
    sq = jnp.sum(dx * dx, axis=0)
    el = jnp.sqrt(sq)

    m = means_ref[...]
    t = el[:, None, :] - m[None, :, :]
    out_ref[0] = jnp.exp2(t * t * _C2)


def _make_ef_call():
    return pl.pallas_call(
        _ef_body,
        grid=(_N, _JS),
        in_specs=[
            pl.BlockSpec((1, 3, _B), lambda i, j: (i, 0, 0)),
            pl.BlockSpec((1, 3, _NJ, _B), lambda i, j: (j, 0, 0, 0)),
            pl.BlockSpec((_EF, 1), lambda i, j: (0, 0)),
        ],
        out_specs=pl.BlockSpec((1, _NJ, _EF, _B), lambda i, j: (i, j, 0, 0)),
        out_shape=jax.ShapeDtypeStruct((_N, _N // _NJ * _NJ, _EF, _B), jnp.float32),
    )


_EF_CALL = _make_ef_call()


def kernel(z, pos, z_emb):
    table = z_emb.at[0].set(0.0)
    z_flat3 = z.astype(jnp.int32).reshape(_NW, _NCH, _CH)

    prow = pos.transpose(1, 2, 0)
    pall = (
        pos.transpose(2, 1, 0)
        .reshape(3, _JS, _NJ, _B)
        .transpose(1, 0, 2, 3)
    )
    means_col = jnp.asarray(_MEANS).reshape(_EF, 1)

    emb_flat = _make_sc_gather()(table, z_flat3)
    ef_t = _EF_CALL(prow, pall, means_col)

    emb1 = emb_flat.reshape(_B, _N, _D)
    ef = ef_t.transpose(3, 0, 1, 2)
    return (emb1, ef)


# speedup vs baseline: 1.3177x; 1.3177x over previous
import functools

import jax
import jax.numpy as jnp
import numpy as np
from jax import lax
from jax.experimental import pallas as pl
from jax.experimental.pallas import tpu as pltpu
from jax.experimental.pallas import tpu_sc as plsc

_B, _N = 256, 64
_D = 256
_EF = 64
_RUP = 5.0

_MEANS = np.linspace(0.0, _RUP, _EF, dtype=np.float32)
_DELTA = np.float32(_MEANS[1] - _MEANS[0])
_COEFF = float(np.float32(0.5) / (_DELTA * _DELTA))


_NW = 32
_TOTAL = _B * _N
_ROWS_W = _TOTAL // _NW
_CH = 128
_NCH = _ROWS_W // _CH


@functools.lru_cache(maxsize=1)
def _make_sc_gather():
    info = plsc.get_sparse_core_info()
    nc, ns = info.num_cores, info.num_subcores
    nw = nc * ns
    assert nw == _NW
    rows_w, ch, nch = _ROWS_W, _CH, _NCH

    mesh = plsc.VectorSubcoreMesh(core_axis_name="c", subcore_axis_name="s")

    @functools.partial(
        pl.kernel,
        mesh=mesh,
        out_type=jax.ShapeDtypeStruct((_TOTAL, _D), jnp.float32),
        scratch_types=[
            pltpu.VMEM((nch, ch), jnp.int32),
            pltpu.VMEM((ch, _D), jnp.float32),
            pltpu.VMEM((ch, _D), jnp.float32),
            pltpu.VMEM((ch, _D), jnp.float32),
            pltpu.SemaphoreType.DMA,
            pltpu.SemaphoreType.DMA,
            pltpu.SemaphoreType.DMA,
            pltpu.SemaphoreType.DMA,
            pltpu.SemaphoreType.DMA,
            pltpu.SemaphoreType.DMA,
        ],
    )
    def sc_gather(table_hbm, idx_hbm, out_hbm, idx_v, buf0, buf1, buf2,
                  gs0, gs1, gs2, os0, os1, os2):
        wid = lax.axis_index("s") * nc + lax.axis_index("c")
        base = wid * rows_w
        pltpu.sync_copy(idx_hbm.at[wid], idx_v)
        bufs = (buf0, buf1, buf2)
        gsems = (gs0, gs1, gs2)
        osems = (os0, os1, os2)
        nbuf = 3
        gathers = [None] * nch
        outs = [None] * nch
        for c in range(min(nbuf, nch)):
            gathers[c] = pltpu.async_copy(
                table_hbm.at[idx_v.at[c]], bufs[c % nbuf], gsems[c % nbuf]
            )
        for c in range(nch):
            gathers[c].wait()
            outs[c] = pltpu.async_copy(
                bufs[c % nbuf], out_hbm.at[pl.ds(base + c * ch, ch)],
                osems[c % nbuf],
            )
            nxt = c + nbuf
            if nxt < nch:
                outs[nxt - nbuf].wait()
                gathers[nxt] = pltpu.async_copy(
                    table_hbm.at[idx_v.at[nxt]], bufs[nxt % nbuf], gsems[nxt % nbuf]
                )
        for c in range(max(0, nch - nbuf), nch):
            outs[c].wait()

    return sc_gather


_C2 = float(-np.float64(_COEFF) * np.log2(np.e))


_JS = 1
_NJ = _N // _JS


def _ef_body(prow_ref, pall_ref, means_ref, out_ref):
    row = prow_ref[0]
    pall = pall_ref[0]

    dx = pall - row[:, None, :]
    sq = jnp.sum(dx * dx, axis=0)
    el = jnp.sqrt(sq)

    m = means_ref[...]
    t = el[:, None, :] - m[None, :, :]
    out_ref[0] = jnp.exp2(t * t * _C2)


def _make_ef_call():
    return pl.pallas_call(
        _ef_body,
        grid=(_N, _JS),
        in_specs=[
            pl.BlockSpec((1, 3, _B), lambda i, j: (i, 0, 0)),
            pl.BlockSpec((1, 3, _NJ, _B), lambda i, j: (j, 0, 0, 0)),
            pl.BlockSpec((_EF, 1), lambda i, j: (0, 0)),
        ],
        out_specs=pl.BlockSpec((1, _NJ, _EF, _B), lambda i, j: (i, j, 0, 0)),
        out_shape=jax.ShapeDtypeStruct((_N, _N // _NJ * _NJ, _EF, _B), jnp.float32),
    )


_EF_CALL = _make_ef_call()


def kernel(z, pos, z_emb):
    table = z_emb.at[0].set(0.0)
    z_flat3 = z.astype(jnp.int32).reshape(_NW, _NCH, _CH)

    prow = pos.transpose(1, 2, 0)
    pall = (
        pos.transpose(2, 1, 0)
        .reshape(3, _JS, _NJ, _B)
        .transpose(1, 0, 2, 3)
    )
    means_col = jnp.asarray(_MEANS).reshape(_EF, 1)

    emb_flat = _make_sc_gather()(table, z_flat3)
    ef_t = _EF_CALL(prow, pall, means_col)

    emb1 = emb_flat.reshape(_B, _N, _D)
    ef = ef_t.transpose(3, 0, 1, 2)
    return (emb1, ef)


# speedup vs baseline: 1.3323x; 1.0111x over previous
import functools

import jax
import jax.numpy as jnp
import numpy as np
from jax import lax
from jax.experimental import pallas as pl
from jax.experimental.pallas import tpu as pltpu
from jax.experimental.pallas import tpu_sc as plsc

_B, _N = 256, 64
_D = 256
_EF = 64
_RUP = 5.0

_MEANS = np.linspace(0.0, _RUP, _EF, dtype=np.float32)
_DELTA = np.float32(_MEANS[1] - _MEANS[0])
_COEFF = float(np.float32(0.5) / (_DELTA * _DELTA))


_NW = 32
_TOTAL = _B * _N
_ROWS_W = _TOTAL // _NW
_CH = 128
_NCH = _ROWS_W // _CH


@functools.lru_cache(maxsize=1)
def _make_sc_gather():
    info = plsc.get_sparse_core_info()
    nc, ns = info.num_cores, info.num_subcores
    nw = nc * ns
    assert nw == _NW
    rows_w, ch, nch = _ROWS_W, _CH, _NCH

    mesh = plsc.VectorSubcoreMesh(core_axis_name="c", subcore_axis_name="s")

    @functools.partial(
        pl.kernel,
        mesh=mesh,
        out_type=jax.ShapeDtypeStruct((_TOTAL, _D), jnp.float32),
        scratch_types=[
            pltpu.VMEM((nch, ch), jnp.int32),
            pltpu.VMEM((ch, _D), jnp.float32),
            pltpu.VMEM((ch, _D), jnp.float32),
            pltpu.VMEM((ch, _D), jnp.float32),
            pltpu.SemaphoreType.DMA,
            pltpu.SemaphoreType.DMA,
            pltpu.SemaphoreType.DMA,
            pltpu.SemaphoreType.DMA,
            pltpu.SemaphoreType.DMA,
            pltpu.SemaphoreType.DMA,
        ],
    )
    def sc_gather(table_hbm, idx_hbm, out_hbm, idx_v, buf0, buf1, buf2,
                  gs0, gs1, gs2, os0, os1, os2):
        wid = lax.axis_index("s") * nc + lax.axis_index("c")
        base = wid * rows_w
        pltpu.sync_copy(idx_hbm.at[wid], idx_v)
        bufs = (buf0, buf1, buf2)
        gsems = (gs0, gs1, gs2)
        osems = (os0, os1, os2)
        nbuf = 3
        gathers = [None] * nch
        outs = [None] * nch
        for c in range(min(nbuf, nch)):
            gathers[c] = pltpu.async_copy(
                table_hbm.at[idx_v.at[c]], bufs[c % nbuf], gsems[c % nbuf]
            )
        for c in range(nch):
            gathers[c].wait()
            outs[c] = pltpu.async_copy(
                bufs[c % nbuf], out_hbm.at[pl.ds(base + c * ch, ch)],
                osems[c % nbuf],
            )
            nxt = c + nbuf
            if nxt < nch:
                outs[nxt - nbuf].wait()
                gathers[nxt] = pltpu.async_copy(
                    table_hbm.at[idx_v.at[nxt]], bufs[nxt % nbuf], gsems[nxt % nbuf]
                )
        for c in range(max(0, nch - nbuf), nch):
            outs[c].wait()

    return sc_gather


_C2 = float(-np.float64(_COEFF) * np.log2(np.e))


_IB = 2


def _ef_body(prow_ref, pall_ref, means_ref, out_ref):
    pall = pall_ref[...]
    m = means_ref[...]
    for r in range(_IB):
        row = prow_ref[r]
        dx = pall - row[:, None, :]
        sq = jnp.sum(dx * dx, axis=0)
        el = jnp.sqrt(sq)
        t = el[:, None, :] - m[None, :, :]
        out_ref[r] = jnp.exp2(t * t * _C2)


def _make_ef_call():
    return pl.pallas_call(
        _ef_body,
        grid=(_N // _IB,),
        in_specs=[
            pl.BlockSpec((_IB, 3, _B), lambda i: (i, 0, 0)),
            pl.BlockSpec((3, _N, _B), lambda i: (0, 0, 0)),
            pl.BlockSpec((_EF, 1), lambda i: (0, 0)),
        ],
        out_specs=pl.BlockSpec((_IB, _N, _EF, _B), lambda i: (i, 0, 0, 0)),
        out_shape=jax.ShapeDtypeStruct((_N, _N, _EF, _B), jnp.float32),
    )


_EF_CALL = _make_ef_call()


def kernel(z, pos, z_emb):
    table = z_emb.at[0].set(0.0)
    z_flat3 = z.astype(jnp.int32).reshape(_NW, _NCH, _CH)

    prow = pos.transpose(1, 2, 0)
    pall = pos.transpose(2, 1, 0)
    means_col = jnp.asarray(_MEANS).reshape(_EF, 1)

    emb_flat = _make_sc_gather()(table, z_flat3)
    ef_t = _EF_CALL(prow, pall, means_col)

    emb1 = emb_flat.reshape(_B, _N, _D)
    ef = ef_t.transpose(3, 0, 1, 2)
    return (emb1, ef)


# speedup vs baseline: 1.3356x; 1.0025x over previous
import functools

import jax
import jax.numpy as jnp
import numpy as np
from jax import lax
from jax.experimental import pallas as pl
from jax.experimental.pallas import tpu as pltpu
from jax.experimental.pallas import tpu_sc as plsc

_B, _N = 256, 64
_D = 256
_EF = 64
_RUP = 5.0

_MEANS = np.linspace(0.0, _RUP, _EF, dtype=np.float32)
_DELTA = np.float32(_MEANS[1] - _MEANS[0])
_COEFF = float(np.float32(0.5) / (_DELTA * _DELTA))


_NW = 32
_TOTAL = _B * _N
_ROWS_W = _TOTAL // _NW
_CH = 128
_NCH = _ROWS_W // _CH


@functools.lru_cache(maxsize=1)
def _make_sc_gather():
    info = plsc.get_sparse_core_info()
    nc, ns = info.num_cores, info.num_subcores
    nw = nc * ns
    assert nw == _NW
    rows_w, ch, nch = _ROWS_W, _CH, _NCH

    mesh = plsc.VectorSubcoreMesh(core_axis_name="c", subcore_axis_name="s")

    @functools.partial(
        pl.kernel,
        mesh=mesh,
        out_type=jax.ShapeDtypeStruct((_TOTAL, _D), jnp.float32),
        scratch_types=[
            pltpu.VMEM((nch, ch), jnp.int32),
            pltpu.VMEM((ch, _D), jnp.float32),
            pltpu.VMEM((ch, _D), jnp.float32),
            pltpu.VMEM((ch, _D), jnp.float32),
            pltpu.SemaphoreType.DMA,
            pltpu.SemaphoreType.DMA,
            pltpu.SemaphoreType.DMA,
            pltpu.SemaphoreType.DMA,
            pltpu.SemaphoreType.DMA,
            pltpu.SemaphoreType.DMA,
        ],
    )
    def sc_gather(table_hbm, idx_hbm, out_hbm, idx_v, buf0, buf1, buf2,
                  gs0, gs1, gs2, os0, os1, os2):
        wid = lax.axis_index("s") * nc + lax.axis_index("c")
        base = wid * rows_w
        pltpu.sync_copy(idx_hbm.at[wid], idx_v)
        bufs = (buf0, buf1, buf2)
        gsems = (gs0, gs1, gs2)
        osems = (os0, os1, os2)
        nbuf = 3
        gathers = [None] * nch
        outs = [None] * nch
        for c in range(min(nbuf, nch)):
            gathers[c] = pltpu.async_copy(
                table_hbm.at[idx_v.at[c]], bufs[c % nbuf], gsems[c % nbuf]
            )
        for c in range(nch):
            gathers[c].wait()
            outs[c] = pltpu.async_copy(
                bufs[c % nbuf], out_hbm.at[pl.ds(base + c * ch, ch)],
                osems[c % nbuf],
            )
            nxt = c + nbuf
            if nxt < nch:
                outs[nxt - nbuf].wait()
                gathers[nxt] = pltpu.async_copy(
                    table_hbm.at[idx_v.at[nxt]], bufs[nxt % nbuf], gsems[nxt % nbuf]
                )
        for c in range(max(0, nch - nbuf), nch):
            outs[c].wait()

    return sc_gather


_C2 = float(-np.float64(_COEFF) * np.log2(np.e))


_IB = 2


def _ef_body(prow_ref, pall_ref, means_ref, out_ref):
    pall = pall_ref[...]
    m = means_ref[...]
    for r in range(_IB):
        row = prow_ref[r]
        dx = pall - row[:, None, :]
        sq = jnp.sum(dx * dx, axis=0)
        el = jnp.sqrt(sq)
        t = el[:, None, :] - m[None, :, :]
        out_ref[r] = jnp.exp2(t * t * _C2)


def _make_ef_call():
    return pl.pallas_call(
        _ef_body,
        grid=(_N // _IB,),
        in_specs=[
            pl.BlockSpec((_IB, 3, _B), lambda i: (i, 0, 0)),
            pl.BlockSpec((3, _N, _B), lambda i: (0, 0, 0)),
            pl.BlockSpec((_EF, 1), lambda i: (0, 0)),
        ],
        out_specs=pl.BlockSpec((_IB, _N, _EF, _B), lambda i: (i, 0, 0, 0)),
        out_shape=jax.ShapeDtypeStruct((_N, _N, _EF, _B), jnp.float32),
        compiler_params=pltpu.CompilerParams(
            dimension_semantics=("parallel",),
        ),
    )


_EF_CALL = _make_ef_call()


def kernel(z, pos, z_emb):
    table = z_emb.at[0].set(0.0)
    z_flat3 = z.astype(jnp.int32).reshape(_NW, _NCH, _CH)

    prow = pos.transpose(1, 2, 0)
    pall = pos.transpose(2, 1, 0)
    means_col = jnp.asarray(_MEANS).reshape(_EF, 1)

    emb_flat = _make_sc_gather()(table, z_flat3)
    ef_t = _EF_CALL(prow, pall, means_col)

    emb1 = emb_flat.reshape(_B, _N, _D)
    ef = ef_t.transpose(3, 0, 1, 2)
    return (emb1, ef)
